# Initial kernel scaffold; baseline (speedup 1.0000x reference)
import jax
import jax.numpy as jnp
from jax.experimental import pallas as pl


def kernel(x, edge_index, W1, b1, W2, b2, W3, b3, W4, b4, Wp, bp):
    raise NotImplementedError("write your pallas kernel here")


# speedup vs baseline: 33.9489x; 33.9489x over previous
import functools

import jax
import jax.numpy as jnp
from jax import lax
from jax.experimental import pallas as pl
from jax.experimental.pallas import tpu as pltpu
from jax.experimental.pallas import tpu_sc as plsc

N = 10000
NP = 10240
E = 640000
D = 128
H = 20
C = 40

NTILES = 32
EPT = E // NTILES
NFG = 4
NEC = 8
FPG = H // NFG
ECH = E // NEC
SUB = 8000
NB = 1024

_mesh = plsc.VectorSubcoreMesh(
    core_axis_name="c", subcore_axis_name="s", num_cores=2, num_subcores=16
)
_sc_params = pltpu.CompilerParams(
    needs_layout_passes=False, use_tc_tiling_on_sc=False
)


def _sc_deg(dst):
    @functools.partial(
        pl.kernel,
        out_type=jax.ShapeDtypeStruct((NTILES, NP), jnp.float32),
        mesh=_mesh,
        compiler_params=_sc_params,
        scratch_types=[
            pltpu.VMEM((EPT,), jnp.int32),
            pltpu.VMEM((NP,), jnp.float32),
        ],
    )
    def k(dst_hbm, out_hbm, dbuf, acc):
        c = lax.axis_index("c")
        s = lax.axis_index("s")
        wid = c * 16 + s

        def zero(i, _):
            acc[pl.ds(i * 16, 16)] = jnp.zeros((16,), jnp.float32)
            return _

        lax.fori_loop(0, NP // 16, zero, 0)

        off = pl.multiple_of(wid * EPT, 8)
        pltpu.sync_copy(dst_hbm.at[pl.ds(off, EPT)], dbuf)
        ones = jnp.full((16,), 1.0, jnp.float32)

        def body(i, _):
            d = dbuf[pl.ds(i * 16, 16)]
            plsc.addupdate_scatter(acc, [d], ones)
            return _

        lax.fori_loop(0, EPT // 16, body, 0)
        pltpu.sync_copy(acc, out_hbm.at[wid])

    return k(dst)


def _sc_edge(g, src, dst):
    @functools.partial(
        pl.kernel,
        out_type=jax.ShapeDtypeStruct((NTILES, FPG, NP), jnp.float32),
        mesh=_mesh,
        compiler_params=_sc_params,
        scratch_types=[
            pltpu.VMEM((FPG, NP), jnp.float32),
            pltpu.VMEM((FPG, NP), jnp.float32),
            pltpu.VMEM((SUB,), jnp.int32),
            pltpu.VMEM((SUB,), jnp.int32),
        ],
    )
    def k(g_hbm, src_hbm, dst_hbm, out_hbm, gbuf, acc, sbuf, dbuf):
        c = lax.axis_index("c")
        s = lax.axis_index("s")
        wid = c * 16 + s
        fg = wid % NFG
        ec = wid // NFG

        pltpu.sync_copy(g_hbm.at[pl.ds(fg * FPG, FPG)], gbuf)

        for j in range(FPG):
            def zero(i, _, j=j):
                acc[j, pl.ds(i * 16, 16)] = jnp.zeros((16,), jnp.float32)
                return _

            lax.fori_loop(0, NP // 16, zero, 0)

        base = ec * ECH
        for sub in range(ECH // SUB):
            off = pl.multiple_of(base + sub * SUB, 8)
            pltpu.sync_copy(src_hbm.at[pl.ds(off, SUB)], sbuf)
            pltpu.sync_copy(dst_hbm.at[pl.ds(off, SUB)], dbuf)

            def body(i, _):
                sv = sbuf[pl.ds(i * 16, 16)]
                dv = dbuf[pl.ds(i * 16, 16)]
                for j in range(FPG):
                    fj = jnp.full((16,), j, jnp.int32)
                    v = plsc.load_gather(gbuf, [fj, sv])
                    plsc.addupdate_scatter(acc, [fj, dv], v)
                return _

            lax.fori_loop(0, SUB // 16, body, 0)

        pltpu.sync_copy(acc, out_hbm.at[wid])

    return k(g, src, dst)


def _agg_from_parts(p):
    groups = []
    for fg in range(NFG):
        t = p[fg]
        for ec in range(1, NEC):
            t = t + p[ec * NFG + fg]
        groups.append(t)
    return jnp.concatenate(groups, axis=0)


def _tc_pre_body(parts_ref, x_ref, w1_ref, dinv_ref, g1_ref):
    deg = jnp.sum(parts_ref[...], axis=0, keepdims=True)
    dinv = jnp.where(deg > 0, lax.rsqrt(jnp.maximum(deg, 1e-12)), 0.0)
    dinv_ref[...] = dinv
    h = lax.dot_general(
        w1_ref[...], x_ref[...], (((0,), (1,)), ((), ())),
        preferred_element_type=jnp.float32,
    )
    g1_ref[...] = h * dinv


def _tc_pre(deg_parts, xp, W1):
    return pl.pallas_call(
        _tc_pre_body,
        grid=(NP // NB,),
        in_specs=[
            pl.BlockSpec((NTILES, NB), lambda i: (0, i)),
            pl.BlockSpec((NB, D), lambda i: (i, 0)),
            pl.BlockSpec((D, H), lambda i: (0, 0)),
        ],
        out_specs=[
            pl.BlockSpec((1, NB), lambda i: (0, i)),
            pl.BlockSpec((H, NB), lambda i: (0, i)),
        ],
        out_shape=[
            jax.ShapeDtypeStruct((1, NP), jnp.float32),
            jax.ShapeDtypeStruct((H, NP), jnp.float32),
        ],
    )(deg_parts, xp, W1)


def _tc_mid_body(parts_ref, dinv_ref, b_ref, wn_ref, h_ref, gn_ref):
    dinv = dinv_ref[...]
    agg = _agg_from_parts(parts_ref[...])
    hl = jnp.maximum(agg * dinv + b_ref[...], 0.0)
    h_ref[...] = hl
    gn = lax.dot_general(
        wn_ref[...], hl, (((0,), (0,)), ((), ())),
        preferred_element_type=jnp.float32,
    )
    gn_ref[...] = gn * dinv


def _tc_mid(parts, dinv, b_col, Wn):
    return pl.pallas_call(
        _tc_mid_body,
        grid=(NP // NB,),
        in_specs=[
            pl.BlockSpec((NTILES, FPG, NB), lambda i: (0, 0, i)),
            pl.BlockSpec((1, NB), lambda i: (0, i)),
            pl.BlockSpec((H, 1), lambda i: (0, 0)),
            pl.BlockSpec((H, H), lambda i: (0, 0)),
        ],
        out_specs=[
            pl.BlockSpec((H, NB), lambda i: (0, i)),
            pl.BlockSpec((H, NB), lambda i: (0, i)),
        ],
        out_shape=[
            jax.ShapeDtypeStruct((H, NP), jnp.float32),
            jax.ShapeDtypeStruct((H, NP), jnp.float32),
        ],
    )(parts, dinv, b_col, Wn)


def _tc_final_body(parts_ref, dinv_ref, b4_ref, h1_ref, h2_ref, h3_ref,
                   wp_ref, bp_ref, out_ref):
    dinv = dinv_ref[...]
    agg = _agg_from_parts(parts_ref[...])
    h4 = jnp.maximum(agg * dinv + b4_ref[...], 0.0)
    cat = jnp.concatenate(
        [h1_ref[...], h2_ref[...], h3_ref[...], h4], axis=0)
    logits = lax.dot_general(
        cat, wp_ref[...], (((0,), (0,)), ((), ())),
        preferred_element_type=jnp.float32,
    ) + bp_ref[...]
    m = jnp.max(logits, axis=1, keepdims=True)
    shifted = logits - m
    lse = jnp.log(jnp.sum(jnp.exp(shifted), axis=1, keepdims=True))
    out_ref[...] = shifted - lse


def _tc_final(parts, dinv, b4_col, h1, h2, h3, Wp, bp_row):
    return pl.pallas_call(
        _tc_final_body,
        grid=(NP // NB,),
        in_specs=[
            pl.BlockSpec((NTILES, FPG, NB), lambda i: (0, 0, i)),
            pl.BlockSpec((1, NB), lambda i: (0, i)),
            pl.BlockSpec((H, 1), lambda i: (0, 0)),
            pl.BlockSpec((H, NB), lambda i: (0, i)),
            pl.BlockSpec((H, NB), lambda i: (0, i)),
            pl.BlockSpec((H, NB), lambda i: (0, i)),
            pl.BlockSpec((4 * H, C), lambda i: (0, 0)),
            pl.BlockSpec((1, C), lambda i: (0, 0)),
        ],
        out_specs=pl.BlockSpec((NB, C), lambda i: (i, 0)),
        out_shape=jax.ShapeDtypeStruct((NP, C), jnp.float32),
    )(parts, dinv, b4_col, h1, h2, h3, Wp, bp_row)


def kernel(x, edge_index, W1, b1, W2, b2, W3, b3, W4, b4, Wp, bp):
    src = edge_index[0]
    dst = edge_index[1]
    xp = jnp.pad(x, ((0, NP - N), (0, 0)))

    deg_parts = _sc_deg(dst)
    dinv, g1 = _tc_pre(deg_parts, xp, W1)

    parts1 = _sc_edge(g1, src, dst)
    h1, g2 = _tc_mid(parts1, dinv, b1[:, None], W2)

    parts2 = _sc_edge(g2, src, dst)
    h2, g3 = _tc_mid(parts2, dinv, b2[:, None], W3)

    parts3 = _sc_edge(g3, src, dst)
    h3, g4 = _tc_mid(parts3, dinv, b3[:, None], W4)

    parts4 = _sc_edge(g4, src, dst)
    out = _tc_final(parts4, dinv, b4[:, None], h1, h2, h3, Wp, bp[None, :])
    return out[:N]


# speedup vs baseline: 63.3195x; 1.8651x over previous
import functools

import jax
import jax.numpy as jnp
from jax import lax
from jax.experimental import pallas as pl
from jax.experimental.pallas import tpu as pltpu
from jax.experimental.pallas import tpu_sc as plsc

N = 10000
NP = 10240
E = 640000
D = 128
H = 20
C = 40

NTILES = 32
EPT = E // NTILES
NFG = 4
NEC = 8
FPG = H // NFG
ECH = E // NEC
SUB = 8000
NB = 1024

_mesh = plsc.VectorSubcoreMesh(
    core_axis_name="c", subcore_axis_name="s", num_cores=2, num_subcores=16
)
_sc_params = pltpu.CompilerParams(
    needs_layout_passes=False, use_tc_tiling_on_sc=False
)


def _sc_deg(dst):
    @functools.partial(
        pl.kernel,
        out_type=jax.ShapeDtypeStruct((NTILES, NP), jnp.float32),
        mesh=_mesh,
        compiler_params=_sc_params,
        scratch_types=[
            pltpu.VMEM((EPT,), jnp.int32),
            pltpu.VMEM((NP,), jnp.float32),
        ],
    )
    def k(dst_hbm, out_hbm, dbuf, acc):
        c = lax.axis_index("c")
        s = lax.axis_index("s")
        wid = c * 16 + s

        @plsc.parallel_loop(0, NP // 16, unroll=4)
        def _(i):
            acc[pl.ds(i * 16, 16)] = jnp.zeros((16,), jnp.float32)

        off = pl.multiple_of(wid * EPT, 8)
        pltpu.sync_copy(dst_hbm.at[pl.ds(off, EPT)], dbuf)
        ones = jnp.full((16,), 1.0, jnp.float32)

        @plsc.parallel_loop(0, EPT // 16, unroll=4)
        def _(i):
            d = dbuf[pl.ds(i * 16, 16)]
            plsc.addupdate_scatter(acc, [d], ones)
        pltpu.sync_copy(acc, out_hbm.at[wid])

    return k(dst)


def _sc_edge(g, src, dst):
    @functools.partial(
        pl.kernel,
        out_type=jax.ShapeDtypeStruct((NTILES, FPG, NP), jnp.float32),
        mesh=_mesh,
        compiler_params=_sc_params,
        scratch_types=[
            pltpu.VMEM((FPG, NP), jnp.float32),
            pltpu.VMEM((FPG, NP), jnp.float32),
            pltpu.VMEM((SUB,), jnp.int32),
            pltpu.VMEM((SUB,), jnp.int32),
        ],
    )
    def k(g_hbm, src_hbm, dst_hbm, out_hbm, gbuf, acc, sbuf, dbuf):
        c = lax.axis_index("c")
        s = lax.axis_index("s")
        wid = c * 16 + s
        fg = wid % NFG
        ec = wid // NFG

        pltpu.sync_copy(g_hbm.at[pl.ds(fg * FPG, FPG)], gbuf)

        for j in range(FPG):
            @plsc.parallel_loop(0, NP // 16, unroll=4)
            def _(i, j=j):
                acc[j, pl.ds(i * 16, 16)] = jnp.zeros((16,), jnp.float32)

        base = ec * ECH
        for sub in range(ECH // SUB):
            off = pl.multiple_of(base + sub * SUB, 8)
            pltpu.sync_copy(src_hbm.at[pl.ds(off, SUB)], sbuf)
            pltpu.sync_copy(dst_hbm.at[pl.ds(off, SUB)], dbuf)

            @plsc.parallel_loop(0, SUB // 16, unroll=2)
            def _(i):
                sv = sbuf[pl.ds(i * 16, 16)]
                dv = dbuf[pl.ds(i * 16, 16)]
                for j in range(FPG):
                    fj = jnp.full((16,), j, jnp.int32)
                    v = plsc.load_gather(gbuf, [fj, sv])
                    plsc.addupdate_scatter(acc, [fj, dv], v)

        pltpu.sync_copy(acc, out_hbm.at[wid])

    return k(g, src, dst)


def _agg_from_parts(p):
    groups = []
    for fg in range(NFG):
        t = p[fg]
        for ec in range(1, NEC):
            t = t + p[ec * NFG + fg]
        groups.append(t)
    return jnp.concatenate(groups, axis=0)


def _tc_pre_body(parts_ref, x_ref, w1_ref, dinv_ref, g1_ref):
    deg = jnp.sum(parts_ref[...], axis=0, keepdims=True)
    dinv = jnp.where(deg > 0, lax.rsqrt(jnp.maximum(deg, 1e-12)), 0.0)
    dinv_ref[...] = dinv
    h = lax.dot_general(
        w1_ref[...], x_ref[...], (((0,), (1,)), ((), ())),
        preferred_element_type=jnp.float32,
    )
    g1_ref[...] = h * dinv


def _tc_pre(deg_parts, xp, W1):
    return pl.pallas_call(
        _tc_pre_body,
        grid=(NP // NB,),
        in_specs=[
            pl.BlockSpec((NTILES, NB), lambda i: (0, i)),
            pl.BlockSpec((NB, D), lambda i: (i, 0)),
            pl.BlockSpec((D, H), lambda i: (0, 0)),
        ],
        out_specs=[
            pl.BlockSpec((1, NB), lambda i: (0, i)),
            pl.BlockSpec((H, NB), lambda i: (0, i)),
        ],
        out_shape=[
            jax.ShapeDtypeStruct((1, NP), jnp.float32),
            jax.ShapeDtypeStruct((H, NP), jnp.float32),
        ],
    )(deg_parts, xp, W1)


def _tc_mid_body(parts_ref, dinv_ref, b_ref, wn_ref, h_ref, gn_ref):
    dinv = dinv_ref[...]
    agg = _agg_from_parts(parts_ref[...])
    hl = jnp.maximum(agg * dinv + b_ref[...], 0.0)
    h_ref[...] = hl
    gn = lax.dot_general(
        wn_ref[...], hl, (((0,), (0,)), ((), ())),
        preferred_element_type=jnp.float32,
    )
    gn_ref[...] = gn * dinv


def _tc_mid(parts, dinv, b_col, Wn):
    return pl.pallas_call(
        _tc_mid_body,
        grid=(NP // NB,),
        in_specs=[
            pl.BlockSpec((NTILES, FPG, NB), lambda i: (0, 0, i)),
            pl.BlockSpec((1, NB), lambda i: (0, i)),
            pl.BlockSpec((H, 1), lambda i: (0, 0)),
            pl.BlockSpec((H, H), lambda i: (0, 0)),
        ],
        out_specs=[
            pl.BlockSpec((H, NB), lambda i: (0, i)),
            pl.BlockSpec((H, NB), lambda i: (0, i)),
        ],
        out_shape=[
            jax.ShapeDtypeStruct((H, NP), jnp.float32),
            jax.ShapeDtypeStruct((H, NP), jnp.float32),
        ],
    )(parts, dinv, b_col, Wn)


def _tc_final_body(parts_ref, dinv_ref, b4_ref, h1_ref, h2_ref, h3_ref,
                   wp_ref, bp_ref, out_ref):
    dinv = dinv_ref[...]
    agg = _agg_from_parts(parts_ref[...])
    h4 = jnp.maximum(agg * dinv + b4_ref[...], 0.0)
    cat = jnp.concatenate(
        [h1_ref[...], h2_ref[...], h3_ref[...], h4], axis=0)
    logits = lax.dot_general(
        cat, wp_ref[...], (((0,), (0,)), ((), ())),
        preferred_element_type=jnp.float32,
    ) + bp_ref[...]
    m = jnp.max(logits, axis=1, keepdims=True)
    shifted = logits - m
    lse = jnp.log(jnp.sum(jnp.exp(shifted), axis=1, keepdims=True))
    out_ref[...] = shifted - lse


def _tc_final(parts, dinv, b4_col, h1, h2, h3, Wp, bp_row):
    return pl.pallas_call(
        _tc_final_body,
        grid=(NP // NB,),
        in_specs=[
            pl.BlockSpec((NTILES, FPG, NB), lambda i: (0, 0, i)),
            pl.BlockSpec((1, NB), lambda i: (0, i)),
            pl.BlockSpec((H, 1), lambda i: (0, 0)),
            pl.BlockSpec((H, NB), lambda i: (0, i)),
            pl.BlockSpec((H, NB), lambda i: (0, i)),
            pl.BlockSpec((H, NB), lambda i: (0, i)),
            pl.BlockSpec((4 * H, C), lambda i: (0, 0)),
            pl.BlockSpec((1, C), lambda i: (0, 0)),
        ],
        out_specs=pl.BlockSpec((NB, C), lambda i: (i, 0)),
        out_shape=jax.ShapeDtypeStruct((NP, C), jnp.float32),
    )(parts, dinv, b4_col, h1, h2, h3, Wp, bp_row)


def kernel(x, edge_index, W1, b1, W2, b2, W3, b3, W4, b4, Wp, bp):
    src = edge_index[0]
    dst = edge_index[1]
    xp = jnp.pad(x, ((0, NP - N), (0, 0)))

    deg_parts = _sc_deg(dst)
    dinv, g1 = _tc_pre(deg_parts, xp, W1)

    parts1 = _sc_edge(g1, src, dst)
    h1, g2 = _tc_mid(parts1, dinv, b1[:, None], W2)

    parts2 = _sc_edge(g2, src, dst)
    h2, g3 = _tc_mid(parts2, dinv, b2[:, None], W3)

    parts3 = _sc_edge(g3, src, dst)
    h3, g4 = _tc_mid(parts3, dinv, b3[:, None], W4)

    parts4 = _sc_edge(g4, src, dst)
    out = _tc_final(parts4, dinv, b4[:, None], h1, h2, h3, Wp, bp[None, :])
    return out[:N]


# speedup vs baseline: 77.8437x; 1.2294x over previous
import functools

import jax
import jax.numpy as jnp
from jax import lax
from jax.experimental import pallas as pl
from jax.experimental.pallas import tpu as pltpu
from jax.experimental.pallas import tpu_sc as plsc

N = 10000
NP = 10240
E = 640000
D = 128
H = 20
C = 40

NTILES = 32
EPT = E // NTILES
NFG = 4
NEC = 8
FPG = H // NFG
ECH = E // NEC
SUB = 3200
QS = NP // 4

_mesh = plsc.VectorSubcoreMesh(
    core_axis_name="c", subcore_axis_name="s", num_cores=2, num_subcores=16
)
_sc_params = pltpu.CompilerParams(
    needs_layout_passes=False, use_tc_tiling_on_sc=False
)


def _sc_deg(dst):
    @functools.partial(
        pl.kernel,
        out_type=jax.ShapeDtypeStruct((NTILES, NP), jnp.float32),
        mesh=_mesh,
        compiler_params=_sc_params,
        scratch_types=[
            pltpu.VMEM((EPT,), jnp.int32),
            pltpu.VMEM((NP,), jnp.float32),
        ],
    )
    def k(dst_hbm, out_hbm, dbuf, acc):
        c = lax.axis_index("c")
        s = lax.axis_index("s")
        wid = c * 16 + s

        @plsc.parallel_loop(0, NP // 16, unroll=4)
        def _(i):
            acc[pl.ds(i * 16, 16)] = jnp.zeros((16,), jnp.float32)

        off = pl.multiple_of(wid * EPT, 8)
        pltpu.sync_copy(dst_hbm.at[pl.ds(off, EPT)], dbuf)
        ones = jnp.full((16,), 1.0, jnp.float32)

        @plsc.parallel_loop(0, EPT // 16, unroll=4)
        def _(i):
            d = dbuf[pl.ds(i * 16, 16)]
            plsc.addupdate_scatter(acc, [d], ones)
        pltpu.sync_copy(acc, out_hbm.at[wid])

    return k(dst)


def _sc_edge(g, src, dst):
    @functools.partial(
        pl.kernel,
        out_type=(
            jax.ShapeDtypeStruct((NTILES, FPG, NP), jnp.float32),
            jax.ShapeDtypeStruct((2, H, NP), jnp.float32),
        ),
        mesh=_mesh,
        compiler_params=_sc_params,
        scratch_types=[
            pltpu.VMEM((FPG, N), jnp.float32),
            pltpu.VMEM((FPG, NP), jnp.float32),
            pltpu.VMEM((2, SUB), jnp.int32),
            pltpu.VMEM((2, SUB), jnp.int32),
            pltpu.SemaphoreType.DMA,
            pltpu.SemaphoreType.DMA,
            pltpu.SemaphoreType.DMA,
        ],
    )
    def k(g_hbm, src_hbm, dst_hbm, raw_hbm, out_hbm, gbuf, acc, sbuf, dbuf,
          sem0, sem1, gsem):
        c = lax.axis_index("c")
        s = lax.axis_index("s")
        wid = c * 16 + s
        fg = wid % NFG
        ec = wid // NFG

        gh = pltpu.async_copy(g_hbm.at[pl.ds(fg * FPG, FPG)], gbuf, gsem)

        base = ec * ECH
        sems = (sem0, sem1)
        nch = ECH // SUB

        def start(ch):
            off = pl.multiple_of(base + ch * SUB, 8)
            b = ch % 2
            hs = pltpu.async_copy(src_hbm.at[pl.ds(off, SUB)], sbuf.at[b],
                                  sems[b])
            hd = pltpu.async_copy(dst_hbm.at[pl.ds(off, SUB)], dbuf.at[b],
                                  sems[b])
            return hs, hd

        pend = start(0)

        for j in range(FPG):
            @plsc.parallel_loop(0, NP // 16, unroll=4)
            def _(i, j=j):
                acc[j, pl.ds(i * 16, 16)] = jnp.zeros((16,), jnp.float32)

        gh.wait()
        for ch in range(nch):
            pend[0].wait()
            pend[1].wait()
            if ch + 1 < nch:
                pend = start(ch + 1)
            b = ch % 2

            @plsc.parallel_loop(0, SUB // 16, unroll=4)
            def _(i, b=b):
                sv = sbuf[b, pl.ds(i * 16, 16)]
                dv = dbuf[b, pl.ds(i * 16, 16)]
                for j in range(FPG):
                    fj = jnp.full((16,), j, jnp.int32)
                    v = plsc.load_gather(gbuf, [fj, sv])
                    plsc.addupdate_scatter(acc, [fj, dv], v)

        pltpu.sync_copy(acc, raw_hbm.at[wid])

        plsc.subcore_barrier()
        fg2 = s % NFG
        q = s // NFG
        qoff = pl.multiple_of(q * QS, 8)
        for e in range(4):
            w = c * 16 + e * NFG + fg2
            pltpu.sync_copy(
                raw_hbm.at[w, :, pl.ds(qoff, QS)], gbuf.at[:, pl.ds(0, QS)]
            )
            for r in range(FPG):
                if e == 0:
                    @plsc.parallel_loop(0, QS // 16, unroll=4)
                    def _(i, r=r):
                        acc[r, pl.ds(i * 16, 16)] = gbuf[r, pl.ds(i * 16, 16)]
                else:
                    @plsc.parallel_loop(0, QS // 16, unroll=4)
                    def _(i, r=r):
                        acc[r, pl.ds(i * 16, 16)] = (
                            acc[r, pl.ds(i * 16, 16)]
                            + gbuf[r, pl.ds(i * 16, 16)]
                        )
        pltpu.sync_copy(
            acc.at[:, pl.ds(0, QS)],
            out_hbm.at[c, pl.ds(fg2 * FPG, FPG), pl.ds(qoff, QS)],
        )

    return k(g, src, dst)[1]


def _tc_pre_body(parts_ref, x_ref, w1_ref, dinv_ref, g1_ref):
    deg = jnp.sum(parts_ref[...], axis=0, keepdims=True)
    dinv = jnp.where(deg > 0, lax.rsqrt(jnp.maximum(deg, 1e-12)), 0.0)
    dinv_ref[...] = dinv
    h = lax.dot_general(
        w1_ref[...], x_ref[...], (((0,), (1,)), ((), ())),
        preferred_element_type=jnp.float32,
    )
    g1_ref[...] = h * dinv[:, :N]


def _tc_pre(deg_parts, x, W1):
    return pl.pallas_call(
        _tc_pre_body,
        out_shape=[
            jax.ShapeDtypeStruct((1, NP), jnp.float32),
            jax.ShapeDtypeStruct((H, N), jnp.float32),
        ],
    )(deg_parts, x, W1)


def _tc_mid_body(parts_ref, dinv_ref, b_ref, wn_ref, h_ref, gn_ref):
    dinv = dinv_ref[...]
    agg = parts_ref[0] + parts_ref[1]
    hl = jnp.maximum(agg * dinv + b_ref[...], 0.0)
    h_ref[...] = hl
    gn = lax.dot_general(
        wn_ref[...], hl[:, :N], (((0,), (0,)), ((), ())),
        preferred_element_type=jnp.float32,
    )
    gn_ref[...] = gn * dinv[:, :N]


def _tc_mid(parts, dinv, b_col, Wn):
    return pl.pallas_call(
        _tc_mid_body,
        out_shape=[
            jax.ShapeDtypeStruct((H, NP), jnp.float32),
            jax.ShapeDtypeStruct((H, N), jnp.float32),
        ],
    )(parts, dinv, b_col, Wn)


def _tc_final_body(parts_ref, dinv_ref, b4_ref, h1_ref, h2_ref, h3_ref,
                   wp_ref, bp_ref, out_ref):
    dinv = dinv_ref[...]
    agg = parts_ref[0] + parts_ref[1]
    h4 = jnp.maximum(agg * dinv + b4_ref[...], 0.0)
    cat = jnp.concatenate(
        [h1_ref[...], h2_ref[...], h3_ref[...], h4], axis=0)
    logits = lax.dot_general(
        cat[:, :N], wp_ref[...], (((0,), (0,)), ((), ())),
        preferred_element_type=jnp.float32,
    ) + bp_ref[...]
    m = jnp.max(logits, axis=1, keepdims=True)
    shifted = logits - m
    lse = jnp.log(jnp.sum(jnp.exp(shifted), axis=1, keepdims=True))
    out_ref[...] = shifted - lse


def _tc_final(parts, dinv, b4_col, h1, h2, h3, Wp, bp_row):
    return pl.pallas_call(
        _tc_final_body,
        out_shape=jax.ShapeDtypeStruct((N, C), jnp.float32),
    )(parts, dinv, b4_col, h1, h2, h3, Wp, bp_row)


def kernel(x, edge_index, W1, b1, W2, b2, W3, b3, W4, b4, Wp, bp):
    src = edge_index[0]
    dst = edge_index[1]

    deg_parts = _sc_deg(dst)
    dinv, g1 = _tc_pre(deg_parts, x, W1)

    parts1 = _sc_edge(g1, src, dst)
    h1, g2 = _tc_mid(parts1, dinv, b1[:, None], W2)

    parts2 = _sc_edge(g2, src, dst)
    h2, g3 = _tc_mid(parts2, dinv, b2[:, None], W3)

    parts3 = _sc_edge(g3, src, dst)
    h3, g4 = _tc_mid(parts3, dinv, b3[:, None], W4)

    parts4 = _sc_edge(g4, src, dst)
    return _tc_final(parts4, dinv, b4[:, None], h1, h2, h3, Wp, bp[None, :])


# speedup vs baseline: 80.1770x; 1.0300x over previous
import functools

import jax
import jax.numpy as jnp
from jax import lax
from jax.experimental import pallas as pl
from jax.experimental.pallas import tpu as pltpu
from jax.experimental.pallas import tpu_sc as plsc

N = 10000
NP = 10240
E = 640000
D = 128
H = 20
C = 40

NTILES = 32
EPT = E // NTILES
NFG = 4
NEC = 8
FPG = H // NFG
ECH = E // NEC
SUB = 3200
QS = NP // 4

_mesh = plsc.VectorSubcoreMesh(
    core_axis_name="c", subcore_axis_name="s", num_cores=2, num_subcores=16
)
_sc_params = pltpu.CompilerParams(
    needs_layout_passes=False, use_tc_tiling_on_sc=False
)


def _sc_deg(dst):
    @functools.partial(
        pl.kernel,
        out_type=jax.ShapeDtypeStruct((NTILES, NP), jnp.float32),
        mesh=_mesh,
        compiler_params=_sc_params,
        scratch_types=[
            pltpu.VMEM((EPT,), jnp.int32),
            pltpu.VMEM((NP,), jnp.float32),
        ],
    )
    def k(dst_hbm, out_hbm, dbuf, acc):
        c = lax.axis_index("c")
        s = lax.axis_index("s")
        wid = c * 16 + s

        @plsc.parallel_loop(0, NP // 16, unroll=4)
        def _(i):
            acc[pl.ds(i * 16, 16)] = jnp.zeros((16,), jnp.float32)

        off = pl.multiple_of(wid * EPT, 8)
        pltpu.sync_copy(dst_hbm.at[pl.ds(off, EPT)], dbuf)
        ones = jnp.full((16,), 1.0, jnp.float32)

        @plsc.parallel_loop(0, EPT // 16, unroll=4)
        def _(i):
            d = dbuf[pl.ds(i * 16, 16)]
            plsc.addupdate_scatter(acc, [d], ones)
        pltpu.sync_copy(acc, out_hbm.at[wid])

    return k(dst)


def _sc_edge(g, src, dst):
    @functools.partial(
        pl.kernel,
        out_type=(
            jax.ShapeDtypeStruct((NTILES, FPG, NP), jnp.float32),
            jax.ShapeDtypeStruct((2, H, NP), jnp.float32),
        ),
        mesh=_mesh,
        compiler_params=_sc_params,
        scratch_types=[
            pltpu.VMEM((FPG, N), jnp.float32),
            pltpu.VMEM((FPG, NP), jnp.float32),
            pltpu.VMEM((2, SUB), jnp.int32),
            pltpu.VMEM((2, SUB), jnp.int32),
            pltpu.SemaphoreType.DMA,
            pltpu.SemaphoreType.DMA,
            pltpu.SemaphoreType.DMA,
        ],
    )
    def k(g_hbm, src_hbm, dst_hbm, raw_hbm, out_hbm, gbuf, acc, sbuf, dbuf,
          sem0, sem1, gsem):
        c = lax.axis_index("c")
        s = lax.axis_index("s")
        wid = c * 16 + s
        fg = wid % NFG
        ec = wid // NFG

        gh = pltpu.async_copy(g_hbm.at[pl.ds(fg * FPG, FPG)], gbuf, gsem)

        base = ec * ECH
        sems = (sem0, sem1)
        nch = ECH // SUB

        def start(ch):
            off = pl.multiple_of(base + ch * SUB, 8)
            b = ch % 2
            hs = pltpu.async_copy(src_hbm.at[pl.ds(off, SUB)], sbuf.at[b],
                                  sems[b])
            hd = pltpu.async_copy(dst_hbm.at[pl.ds(off, SUB)], dbuf.at[b],
                                  sems[b])
            return hs, hd

        pend = start(0)

        for j in range(FPG):
            @plsc.parallel_loop(0, NP // 16, unroll=4)
            def _(i, j=j):
                acc[j, pl.ds(i * 16, 16)] = jnp.zeros((16,), jnp.float32)

        gh.wait()
        for ch in range(nch):
            pend[0].wait()
            pend[1].wait()
            if ch + 1 < nch:
                pend = start(ch + 1)
            b = ch % 2

            @plsc.parallel_loop(0, SUB // 16, unroll=4)
            def _(i, b=b):
                sv = sbuf[b, pl.ds(i * 16, 16)]
                dv = dbuf[b, pl.ds(i * 16, 16)]
                for j in range(FPG):
                    fj = jnp.full((16,), j, jnp.int32)
                    v = plsc.load_gather(gbuf, [fj, sv])
                    plsc.addupdate_scatter(acc, [fj, dv], v)

        pltpu.sync_copy(acc, raw_hbm.at[wid])

        plsc.subcore_barrier()
        fg2 = s % NFG
        q = s // NFG
        qoff = pl.multiple_of(q * QS, 8)

        def slab(e):
            return raw_hbm.at[c * 16 + e * NFG + fg2, :, pl.ds(qoff, QS)]

        h0 = pltpu.async_copy(slab(0), acc.at[:, pl.ds(0, QS)], gsem)
        pend_r = pltpu.async_copy(slab(1), gbuf.at[:, pl.ds(0, QS)], sem0)
        h0.wait()
        for e in (1, 2, 3):
            boff = ((e - 1) % 2) * QS
            pend_r.wait()
            if e < 3:
                pend_r = pltpu.async_copy(
                    slab(e + 1), gbuf.at[:, pl.ds((e % 2) * QS, QS)],
                    sems[e % 2])
            for r in range(FPG):
                @plsc.parallel_loop(0, QS // 16, unroll=4)
                def _(i, r=r, boff=boff):
                    acc[r, pl.ds(i * 16, 16)] = (
                        acc[r, pl.ds(i * 16, 16)]
                        + gbuf[r, pl.ds(boff + i * 16, 16)]
                    )
        pltpu.sync_copy(
            acc.at[:, pl.ds(0, QS)],
            out_hbm.at[c, pl.ds(fg2 * FPG, FPG), pl.ds(qoff, QS)],
        )

    return k(g, src, dst)[1]


def _tc_pre_body(parts_ref, x_ref, w1_ref, dinv_ref, g1_ref):
    deg = jnp.sum(parts_ref[...], axis=0, keepdims=True)
    dinv = jnp.where(deg > 0, lax.rsqrt(jnp.maximum(deg, 1e-12)), 0.0)
    dinv_ref[...] = dinv
    h = lax.dot_general(
        w1_ref[...], x_ref[...], (((0,), (1,)), ((), ())),
        preferred_element_type=jnp.float32,
    )
    g1_ref[...] = h * dinv[:, :N]


def _tc_pre(deg_parts, x, W1):
    return pl.pallas_call(
        _tc_pre_body,
        out_shape=[
            jax.ShapeDtypeStruct((1, NP), jnp.float32),
            jax.ShapeDtypeStruct((H, N), jnp.float32),
        ],
    )(deg_parts, x, W1)


def _tc_mid_body(parts_ref, dinv_ref, b_ref, wn_ref, h_ref, gn_ref):
    dinv = dinv_ref[...]
    agg = parts_ref[0] + parts_ref[1]
    hl = jnp.maximum(agg * dinv + b_ref[...], 0.0)
    h_ref[...] = hl
    gn = lax.dot_general(
        wn_ref[...], hl[:, :N], (((0,), (0,)), ((), ())),
        preferred_element_type=jnp.float32,
    )
    gn_ref[...] = gn * dinv[:, :N]


def _tc_mid(parts, dinv, b_col, Wn):
    return pl.pallas_call(
        _tc_mid_body,
        out_shape=[
            jax.ShapeDtypeStruct((H, NP), jnp.float32),
            jax.ShapeDtypeStruct((H, N), jnp.float32),
        ],
    )(parts, dinv, b_col, Wn)


def _tc_final_body(parts_ref, dinv_ref, b4_ref, h1_ref, h2_ref, h3_ref,
                   wp_ref, bp_ref, out_ref):
    dinv = dinv_ref[...]
    agg = parts_ref[0] + parts_ref[1]
    h4 = jnp.maximum(agg * dinv + b4_ref[...], 0.0)
    cat = jnp.concatenate(
        [h1_ref[...], h2_ref[...], h3_ref[...], h4], axis=0)
    logits = lax.dot_general(
        cat[:, :N], wp_ref[...], (((0,), (0,)), ((), ())),
        preferred_element_type=jnp.float32,
    ) + bp_ref[...]
    m = jnp.max(logits, axis=1, keepdims=True)
    shifted = logits - m
    lse = jnp.log(jnp.sum(jnp.exp(shifted), axis=1, keepdims=True))
    out_ref[...] = shifted - lse


def _tc_final(parts, dinv, b4_col, h1, h2, h3, Wp, bp_row):
    return pl.pallas_call(
        _tc_final_body,
        out_shape=jax.ShapeDtypeStruct((N, C), jnp.float32),
    )(parts, dinv, b4_col, h1, h2, h3, Wp, bp_row)


def kernel(x, edge_index, W1, b1, W2, b2, W3, b3, W4, b4, Wp, bp):
    src = edge_index[0]
    dst = edge_index[1]

    deg_parts = _sc_deg(dst)
    dinv, g1 = _tc_pre(deg_parts, x, W1)

    parts1 = _sc_edge(g1, src, dst)
    h1, g2 = _tc_mid(parts1, dinv, b1[:, None], W2)

    parts2 = _sc_edge(g2, src, dst)
    h2, g3 = _tc_mid(parts2, dinv, b2[:, None], W3)

    parts3 = _sc_edge(g3, src, dst)
    h3, g4 = _tc_mid(parts3, dinv, b3[:, None], W4)

    parts4 = _sc_edge(g4, src, dst)
    return _tc_final(parts4, dinv, b4[:, None], h1, h2, h3, Wp, bp[None, :])


# speedup vs baseline: 81.7179x; 1.0192x over previous
import functools

import jax
import jax.numpy as jnp
from jax import lax
from jax.experimental import pallas as pl
from jax.experimental.pallas import tpu as pltpu
from jax.experimental.pallas import tpu_sc as plsc

N = 10000
NP = 10240
E = 640000
D = 128
H = 20
C = 40

NTILES = 32
EPT = E // NTILES
NFG = 4
NEC = 8
FPG = H // NFG
ECH = E // NEC
SUB = 3200
QS = NP // 4

_mesh = plsc.VectorSubcoreMesh(
    core_axis_name="c", subcore_axis_name="s", num_cores=2, num_subcores=16
)
_sc_params = pltpu.CompilerParams(
    needs_layout_passes=False, use_tc_tiling_on_sc=False
)


def _sc_deg(dst):
    @functools.partial(
        pl.kernel,
        out_type=jax.ShapeDtypeStruct((NTILES, NP), jnp.float32),
        mesh=_mesh,
        compiler_params=_sc_params,
        scratch_types=[
            pltpu.VMEM((EPT,), jnp.int32),
            pltpu.VMEM((NP,), jnp.float32),
        ],
    )
    def k(dst_hbm, out_hbm, dbuf, acc):
        c = lax.axis_index("c")
        s = lax.axis_index("s")
        wid = c * 16 + s

        @plsc.parallel_loop(0, NP // 16, unroll=4)
        def _(i):
            acc[pl.ds(i * 16, 16)] = jnp.zeros((16,), jnp.float32)

        off = pl.multiple_of(wid * EPT, 8)
        pltpu.sync_copy(dst_hbm.at[pl.ds(off, EPT)], dbuf)
        ones = jnp.full((16,), 1.0, jnp.float32)

        @plsc.parallel_loop(0, EPT // 16, unroll=4)
        def _(i):
            d = dbuf[pl.ds(i * 16, 16)]
            plsc.addupdate_scatter(acc, [d], ones)
        pltpu.sync_copy(acc, out_hbm.at[wid])

    return k(dst)


def _sc_edge(g, src, dst):
    @functools.partial(
        pl.kernel,
        out_type=(
            jax.ShapeDtypeStruct((NTILES, FPG, NP), jnp.float32),
            jax.ShapeDtypeStruct((2, H, NP), jnp.float32),
        ),
        mesh=_mesh,
        compiler_params=_sc_params,
        scratch_types=[
            pltpu.VMEM((FPG, N), jnp.float32),
            pltpu.VMEM((FPG, NP), jnp.float32),
            pltpu.VMEM((2, SUB), jnp.int32),
            pltpu.VMEM((2, SUB), jnp.int32),
            pltpu.SemaphoreType.DMA,
            pltpu.SemaphoreType.DMA,
            pltpu.SemaphoreType.DMA,
        ],
    )
    def k(g_hbm, src_hbm, dst_hbm, raw_hbm, out_hbm, gbuf, acc, sbuf, dbuf,
          sem0, sem1, gsem):
        c = lax.axis_index("c")
        s = lax.axis_index("s")
        wid = c * 16 + s
        fg = wid % NFG
        ec = wid // NFG

        gh = pltpu.async_copy(g_hbm.at[pl.ds(fg * FPG, FPG)], gbuf, gsem)

        base = ec * ECH
        sems = (sem0, sem1)
        nch = ECH // SUB

        def start(ch):
            off = pl.multiple_of(base + ch * SUB, 8)
            b = ch % 2
            hs = pltpu.async_copy(src_hbm.at[pl.ds(off, SUB)], sbuf.at[b],
                                  sems[b])
            hd = pltpu.async_copy(dst_hbm.at[pl.ds(off, SUB)], dbuf.at[b],
                                  sems[b])
            return hs, hd

        pend = start(0)

        for j in range(FPG):
            @plsc.parallel_loop(0, NP // 16, unroll=4)
            def _(i, j=j):
                acc[j, pl.ds(i * 16, 16)] = jnp.zeros((16,), jnp.float32)

        gh.wait()
        for ch in range(nch):
            pend[0].wait()
            pend[1].wait()
            if ch + 1 < nch:
                pend = start(ch + 1)
            b = ch % 2

            @plsc.parallel_loop(0, SUB // 16, unroll=2)
            def _(i, b=b):
                sv = sbuf[b, pl.ds(i * 16, 16)]
                dv = dbuf[b, pl.ds(i * 16, 16)]
                for j in range(FPG):
                    fj = jnp.full((16,), j, jnp.int32)
                    v = plsc.load_gather(gbuf, [fj, sv])
                    plsc.addupdate_scatter(acc, [fj, dv], v)

        pltpu.sync_copy(acc, raw_hbm.at[wid])

        plsc.subcore_barrier()
        fg2 = s % NFG
        q = s // NFG
        qoff = pl.multiple_of(q * QS, 8)

        def slab(e):
            return raw_hbm.at[c * 16 + e * NFG + fg2, :, pl.ds(qoff, QS)]

        h0 = pltpu.async_copy(slab(0), acc.at[:, pl.ds(0, QS)], gsem)
        pend_r = pltpu.async_copy(slab(1), gbuf.at[:, pl.ds(0, QS)], sem0)
        h0.wait()
        for e in (1, 2, 3):
            boff = ((e - 1) % 2) * QS
            pend_r.wait()
            if e < 3:
                pend_r = pltpu.async_copy(
                    slab(e + 1), gbuf.at[:, pl.ds((e % 2) * QS, QS)],
                    sems[e % 2])
            for r in range(FPG):
                @plsc.parallel_loop(0, QS // 16, unroll=4)
                def _(i, r=r, boff=boff):
                    acc[r, pl.ds(i * 16, 16)] = (
                        acc[r, pl.ds(i * 16, 16)]
                        + gbuf[r, pl.ds(boff + i * 16, 16)]
                    )
        pltpu.sync_copy(
            acc.at[:, pl.ds(0, QS)],
            out_hbm.at[c, pl.ds(fg2 * FPG, FPG), pl.ds(qoff, QS)],
        )

    return k(g, src, dst)[1]


def _tc_pre_body(parts_ref, x_ref, w1_ref, dinv_ref, g1_ref):
    deg = jnp.sum(parts_ref[...], axis=0, keepdims=True)
    dinv = jnp.where(deg > 0, lax.rsqrt(jnp.maximum(deg, 1e-12)), 0.0)
    dinv_ref[...] = dinv
    h = lax.dot_general(
        w1_ref[...], x_ref[...], (((0,), (1,)), ((), ())),
        preferred_element_type=jnp.float32,
    )
    g1_ref[...] = h * dinv[:, :N]


def _tc_pre(deg_parts, x, W1):
    return pl.pallas_call(
        _tc_pre_body,
        out_shape=[
            jax.ShapeDtypeStruct((1, NP), jnp.float32),
            jax.ShapeDtypeStruct((H, N), jnp.float32),
        ],
    )(deg_parts, x, W1)


def _tc_mid_body(parts_ref, dinv_ref, b_ref, wn_ref, h_ref, gn_ref):
    dinv = dinv_ref[...]
    agg = parts_ref[0] + parts_ref[1]
    hl = jnp.maximum(agg * dinv + b_ref[...], 0.0)
    h_ref[...] = hl
    gn = lax.dot_general(
        wn_ref[...], hl[:, :N], (((0,), (0,)), ((), ())),
        preferred_element_type=jnp.float32,
    )
    gn_ref[...] = gn * dinv[:, :N]


def _tc_mid(parts, dinv, b_col, Wn):
    return pl.pallas_call(
        _tc_mid_body,
        out_shape=[
            jax.ShapeDtypeStruct((H, NP), jnp.float32),
            jax.ShapeDtypeStruct((H, N), jnp.float32),
        ],
    )(parts, dinv, b_col, Wn)


def _tc_final_body(parts_ref, dinv_ref, b4_ref, h1_ref, h2_ref, h3_ref,
                   wp_ref, bp_ref, out_ref):
    dinv = dinv_ref[...]
    agg = parts_ref[0] + parts_ref[1]
    h4 = jnp.maximum(agg * dinv + b4_ref[...], 0.0)
    cat = jnp.concatenate(
        [h1_ref[...], h2_ref[...], h3_ref[...], h4], axis=0)
    logits = lax.dot_general(
        cat[:, :N], wp_ref[...], (((0,), (0,)), ((), ())),
        preferred_element_type=jnp.float32,
    ) + bp_ref[...]
    m = jnp.max(logits, axis=1, keepdims=True)
    shifted = logits - m
    lse = jnp.log(jnp.sum(jnp.exp(shifted), axis=1, keepdims=True))
    out_ref[...] = shifted - lse


def _tc_final(parts, dinv, b4_col, h1, h2, h3, Wp, bp_row):
    return pl.pallas_call(
        _tc_final_body,
        out_shape=jax.ShapeDtypeStruct((N, C), jnp.float32),
    )(parts, dinv, b4_col, h1, h2, h3, Wp, bp_row)


def kernel(x, edge_index, W1, b1, W2, b2, W3, b3, W4, b4, Wp, bp):
    src = edge_index[0]
    dst = edge_index[1]

    deg_parts = _sc_deg(dst)
    dinv, g1 = _tc_pre(deg_parts, x, W1)

    parts1 = _sc_edge(g1, src, dst)
    h1, g2 = _tc_mid(parts1, dinv, b1[:, None], W2)

    parts2 = _sc_edge(g2, src, dst)
    h2, g3 = _tc_mid(parts2, dinv, b2[:, None], W3)

    parts3 = _sc_edge(g3, src, dst)
    h3, g4 = _tc_mid(parts3, dinv, b3[:, None], W4)

    parts4 = _sc_edge(g4, src, dst)
    return _tc_final(parts4, dinv, b4[:, None], h1, h2, h3, Wp, bp[None, :])
